# Initial kernel scaffold; baseline (speedup 1.0000x reference)
import jax
import jax.numpy as jnp
from jax.experimental import pallas as pl


def kernel(x, edge_index, edge_attr, W1, b1, W2, b2):
    raise NotImplementedError("write your pallas kernel here")


# speedup vs baseline: 20.9374x; 20.9374x over previous
import jax
import jax.numpy as jnp
from jax import lax
from jax.experimental import pallas as pl
from jax.experimental.pallas import tpu as pltpu
from jax.experimental.pallas import tpu_sc as plsc


def _z():
    return jnp.int32(0)


N = 100000
E = 6400000
ACC_SCALE = 0.02
MAX_VEL = 0.02
NOISE = 0.002

NC = 2
NS = 16
BLK = 128
NBLK = E // BLK
ITERS = -(-NBLK // (NC * NS))
NPAD = 100096
ROWS_PER_TILE = NPAD // NS
CHUNK = 782
NCHUNK = ROWS_PER_TILE // CHUNK


def _u_body(x_ref, w_ref, b_ref, o_ref):
    o_ref[...] = jnp.dot(x_ref[:, :8], w_ref[...],
                         preferred_element_type=jnp.float32) + b_ref[...]


def _compute_u(x, w1a, b1):
    bn = 10000
    return pl.pallas_call(
        _u_body,
        grid=(N // bn,),
        in_specs=[pl.BlockSpec((bn, 11), lambda i: (i, _z())),
                  pl.BlockSpec((8, 16), lambda i: (_z(), _z())),
                  pl.BlockSpec((1, 16), lambda i: (_z(), _z()))],
        out_specs=pl.BlockSpec((bn, 16), lambda i: (i, _z())),
        out_shape=jax.ShapeDtypeStruct((N, 16), jnp.float32),
    )(x, w1a, b1)


def _sc_body(u_hbm, ea_hbm, src_hbm, dst_hbm, w1b_hbm, zeros_hbm, out_hbm,
             agg_sh, srcb, dstb, eab, ug, mb, wv, bounce, sem):
    c = lax.axis_index("c")
    s = lax.axis_index("s")
    w = s * NC + c

    row0 = s * ROWS_PER_TILE
    pltpu.sync_copy(zeros_hbm, bounce)
    for k in range(NCHUNK):
        pltpu.sync_copy(bounce, agg_sh.at[pl.ds(row0 + k * CHUNK, CHUNK)])
    pltpu.sync_copy(w1b_hbm, wv)
    plsc.subcore_barrier()

    w0 = wv[0, :]
    w1 = wv[1, :]
    w2 = wv[2, :]
    w3 = wv[3, :]

    def body(i, carry):
        g = w + jnp.int32(NC * NS) * i

        @pl.when(g < NBLK)
        def _():
            off = g * BLK
            pltpu.sync_copy(src_hbm.at[pl.ds(off, BLK)], srcb)
            pltpu.sync_copy(dst_hbm.at[pl.ds(off, BLK)], dstb)
            for f in range(4):
                pltpu.sync_copy(ea_hbm.at[pl.ds(f * E + off, BLK)],
                                eab.at[pl.ds(f * BLK, BLK)])
            pltpu.async_copy(u_hbm.at[srcb], ug, sem).wait()
            for j in range(BLK):
                z = ug[j, :]
                z = z + plsc.load_gather(eab, [jnp.full((16,), j, jnp.int32)]) * w0
                z = z + plsc.load_gather(eab, [jnp.full((16,), BLK + j, jnp.int32)]) * w1
                z = z + plsc.load_gather(eab, [jnp.full((16,), 2 * BLK + j, jnp.int32)]) * w2
                z = z + plsc.load_gather(eab, [jnp.full((16,), 3 * BLK + j, jnp.int32)]) * w3
                e = jnp.exp(2.0 * z)
                mb[j, :] = 1.0 - 2.0 / (e + 1.0)
            pltpu.sync_copy(mb, agg_sh.at[dstb], add=True)

        return carry

    lax.fori_loop(jnp.int32(0), jnp.int32(ITERS), body, jnp.int32(0))
    plsc.subcore_barrier()

    for k in range(NCHUNK):
        pltpu.sync_copy(agg_sh.at[pl.ds(row0 + k * CHUNK, CHUNK)], bounce)
        pltpu.sync_copy(bounce, out_hbm.at[c, pl.ds(row0 + k * CHUNK, CHUNK)])


_sc_segment = pl.kernel(
    _sc_body,
    out_type=jax.ShapeDtypeStruct((NC, NPAD, 16), jnp.float32),
    mesh=plsc.VectorSubcoreMesh(core_axis_name="c", subcore_axis_name="s"),
    compiler_params=pltpu.CompilerParams(use_tc_tiling_on_sc=False,
                                         needs_layout_passes=False),
    scratch_types=[
        pltpu.VMEM_SHARED((NPAD, 16), jnp.float32),
        pltpu.VMEM((BLK,), jnp.int32),
        pltpu.VMEM((BLK,), jnp.int32),
        pltpu.VMEM((4 * BLK,), jnp.float32),
        pltpu.VMEM((BLK, 16), jnp.float32),
        pltpu.VMEM((BLK, 16), jnp.float32),
        pltpu.VMEM((4, 16), jnp.float32),
        pltpu.VMEM((CHUNK, 16), jnp.float32),
        pltpu.SemaphoreType.DMA,
    ],
)


def _upd_body(agg2_ref, x_ref, nz_ref, w2_ref, b2_ref, o_ref):
    agg = agg2_ref[0] + agg2_ref[1]
    h = jnp.tanh(jnp.dot(agg, w2_ref[...],
                         preferred_element_type=jnp.float32) + b2_ref[...])
    x = x_ref[...]
    col4 = x[:, 4]
    c_mask = col4 == 1.0
    moveable = jnp.logical_or(c_mask, col4 == 2.0)
    cm = c_mask.astype(jnp.float32)
    h = h * moveable.astype(jnp.float32)[:, None]
    acc = h[:, :2] * ACC_SCALE
    vel = jnp.clip(x[:, 2:4] + acc, jnp.float32(-MAX_VEL), jnp.float32(MAX_VEL))
    pos = x[:, :2] + vel
    pos = jnp.where(pos > 1.0, pos - 2.0, pos)
    pos = jnp.where(pos < -1.0, pos + 2.0, pos)
    new_v = jnp.where(moveable[:, None], vel, x[:, 2:4]) + nz_ref[...] * cm[:, None]
    new_p = jnp.where(moveable[:, None], pos, x[:, :2])
    energy = jnp.minimum(x[:, 5] - cm, 10.0)
    o_ref[...] = jnp.concatenate(
        [new_p, new_v, x[:, 4:5], energy[:, None], h[:, 2:]], axis=1)


def _update(agg2, x, nz, w2, b2):
    bn = 2000
    return pl.pallas_call(
        _upd_body,
        grid=(N // bn,),
        in_specs=[pl.BlockSpec((2, bn, 8), lambda i: (_z(), i, _z())),
                  pl.BlockSpec((bn, 11), lambda i: (i, _z())),
                  pl.BlockSpec((bn, 2), lambda i: (i, _z())),
                  pl.BlockSpec((8, 7), lambda i: (_z(), _z())),
                  pl.BlockSpec((1, 7), lambda i: (_z(), _z()))],
        out_specs=pl.BlockSpec((bn, 11), lambda i: (i, _z())),
        out_shape=jax.ShapeDtypeStruct((N, 11), jnp.float32),
    )(agg2, x, nz, w2, b2)


def kernel(x, edge_index, edge_attr, W1, b1, W2, b2):
    src = edge_index[0].astype(jnp.int32)
    dst = edge_index[1].astype(jnp.int32)
    x = x.astype(jnp.float32)
    edge_attr = edge_attr.astype(jnp.float32)
    W1 = W1.astype(jnp.float32)
    b1 = b1.astype(jnp.float32)
    W2 = W2.astype(jnp.float32)
    b2 = b2.astype(jnp.float32)
    pad8 = jnp.zeros((8,), jnp.float32)
    w1a = jnp.concatenate([W1[:8], jnp.zeros((8, 8), jnp.float32)], axis=1)
    w1b = jnp.concatenate([W1[8:], jnp.zeros((4, 8), jnp.float32)], axis=1)
    b1p = jnp.concatenate([b1, pad8])

    u = _compute_u(x, w1a, b1p[None, :])
    ea_flat = edge_attr.T.reshape(4 * E)
    zeros = jnp.zeros((CHUNK, 16), jnp.float32)

    agg2 = _sc_segment(u, ea_flat, src, dst, w1b, zeros)[:, :N, :8]

    nk = jax.random.key(42)
    xn = (jax.random.uniform(jax.random.fold_in(nk, 0), (N,), dtype=jnp.float32) * 2.0 - 1.0) * NOISE
    yn = (jax.random.uniform(jax.random.fold_in(nk, 1), (N,), dtype=jnp.float32) * 2.0 - 1.0) * NOISE
    upd = (jax.random.uniform(jax.random.fold_in(nk, 2), (N,), dtype=jnp.float32) > 0.5).astype(jnp.float32)
    nz = jnp.stack([xn * upd, yn * upd], axis=1)

    return _update(agg2, x, nz, W2, b2[None, :]).astype(jnp.float64)


# speedup vs baseline: 83.2378x; 3.9755x over previous
import jax
import jax.numpy as jnp
from jax import lax
from jax.experimental import pallas as pl
from jax.experimental.pallas import tpu as pltpu
from jax.experimental.pallas import tpu_sc as plsc


def _z():
    return jnp.int32(0)


N = 100000
E = 6400000
ACC_SCALE = 0.02
MAX_VEL = 0.02
NOISE = 0.002

NC = 2
NS = 16
BLK = 256
NW = NC * NS
NBLK = E // BLK
ITER2 = -(-NBLK // (2 * NW))
NPAD = 100096
ROWS_PER_TILE = NPAD // NS
CHUNK = 184
NCHUNK = ROWS_PER_TILE // CHUNK


def _u_body(x_ref, w_ref, b_ref, o_ref):
    o_ref[...] = jnp.dot(x_ref[:, :8], w_ref[...],
                         preferred_element_type=jnp.float32) + b_ref[...]


def _compute_u(x, w1a, b1):
    bn = 10000
    return pl.pallas_call(
        _u_body,
        grid=(N // bn,),
        in_specs=[pl.BlockSpec((bn, 11), lambda i: (i, _z())),
                  pl.BlockSpec((8, 16), lambda i: (_z(), _z())),
                  pl.BlockSpec((1, 16), lambda i: (_z(), _z()))],
        out_specs=pl.BlockSpec((bn, 16), lambda i: (i, _z())),
        out_shape=jax.ShapeDtypeStruct((N, 16), jnp.float32),
    )(x, w1a, b1)


def _sc_body(u_hbm, ea_hbm, src_hbm, dst_hbm, w1b_hbm, zeros_hbm, out_hbm,
             agg_sh,
             srcb0, srcb1, dstA0, dstB0, dstA1, dstB1, eab0, eab1,
             ug0, ug1, wv, bounce,
             sl0, sl1, sg0, sg1):
    c = lax.axis_index("c")
    s = lax.axis_index("s")
    w = s * NC + c

    row0 = s * ROWS_PER_TILE
    pltpu.sync_copy(zeros_hbm, bounce)
    for k in range(NCHUNK):
        pltpu.sync_copy(bounce, agg_sh.at[pl.ds(row0 + k * CHUNK, CHUNK)])
    pltpu.sync_copy(w1b_hbm, wv)
    plsc.subcore_barrier()

    w0 = wv[0, :]
    w1 = wv[1, :]
    w2 = wv[2, :]
    w3 = wv[3, :]

    bufs = ((srcb0, dstA0, dstB0, eab0, ug0, sl0, sg0),
            (srcb1, dstA1, dstB1, eab1, ug1, sl1, sg1))

    def issue_lin(g, b):
        srcb, dstA, dstB, eab, ug, sl, sg = bufs[b]

        @pl.when(g < NBLK)
        def _():
            off = g * BLK
            pltpu.async_copy(src_hbm.at[pl.ds(off, BLK)], srcb, sl)
            pltpu.async_copy(dst_hbm.at[pl.ds(off, 128)], dstA, sl)
            pltpu.async_copy(dst_hbm.at[pl.ds(off + 128, 128)], dstB, sl)
            for f in range(4):
                pltpu.async_copy(ea_hbm.at[pl.ds(f * E + off, BLK)],
                                 eab.at[pl.ds(f * BLK, BLK)], sl)

    def wait_lin_issue_gather(g, b):
        srcb, dstA, dstB, eab, ug, sl, sg = bufs[b]

        @pl.when(g < NBLK)
        def _():
            off = g * BLK
            pltpu.make_async_copy(src_hbm.at[pl.ds(off, BLK)], srcb, sl).wait()
            pltpu.make_async_copy(dst_hbm.at[pl.ds(off, 128)], dstA, sl).wait()
            pltpu.make_async_copy(dst_hbm.at[pl.ds(off + 128, 128)], dstB, sl).wait()
            for f in range(4):
                pltpu.make_async_copy(ea_hbm.at[pl.ds(f * E + off, BLK)],
                                      eab.at[pl.ds(f * BLK, BLK)], sl).wait()
            pltpu.async_copy(u_hbm.at[srcb.at[pl.ds(0, 128)]],
                             ug.at[pl.ds(0, 128)], sg)
            pltpu.async_copy(u_hbm.at[srcb.at[pl.ds(128, 128)]],
                             ug.at[pl.ds(128, 128)], sg)

    def compute_scatter(g, b):
        srcb, dstA, dstB, eab, ug, sl, sg = bufs[b]

        @pl.when(g < NBLK)
        def _():
            pltpu.make_async_copy(u_hbm.at[srcb.at[pl.ds(0, 128)]],
                                  ug.at[pl.ds(0, 128)], sg).wait()
            pltpu.make_async_copy(u_hbm.at[srcb.at[pl.ds(128, 128)]],
                                  ug.at[pl.ds(128, 128)], sg).wait()
            @plsc.parallel_loop(jnp.int32(0), jnp.int32(BLK),
                                step=jnp.int32(16), unroll=2)
            def _grp(base):
                ea0 = eab[pl.ds(base, 16)]
                ea1 = eab[pl.ds(base + BLK, 16)]
                ea2 = eab[pl.ds(base + 2 * BLK, 16)]
                ea3 = eab[pl.ds(base + 3 * BLK, 16)]
                for jj in range(16):
                    idx = jnp.full((16,), jj, jnp.int32)
                    z = ug[base + jj, :]
                    z = z + ea0.at[idx].get(mode="promise_in_bounds") * w0
                    z = z + ea1.at[idx].get(mode="promise_in_bounds") * w1
                    z = z + ea2.at[idx].get(mode="promise_in_bounds") * w2
                    z = z + ea3.at[idx].get(mode="promise_in_bounds") * w3
                    e = jnp.exp(2.0 * z)
                    ug[base + jj, :] = 1.0 - 2.0 / (e + 1.0)
            pltpu.sync_copy(ug.at[pl.ds(0, 128)], agg_sh.at[dstA], add=True)
            pltpu.sync_copy(ug.at[pl.ds(128, 128)], agg_sh.at[dstB], add=True)

    def gblock(i2, b):
        return w + jnp.int32(2 * NW) * i2 + jnp.int32(b * NW)

    issue_lin(gblock(jnp.int32(0), 0), 0)

    def body(i2, carry):
        for b in (0, 1):
            g = gblock(i2, b)
            wait_lin_issue_gather(g, b)
            if b == 0:
                issue_lin(gblock(i2, 1), 1)
            else:
                issue_lin(gblock(i2 + 1, 0), 0)
            compute_scatter(g, b)
        return carry

    lax.fori_loop(jnp.int32(0), jnp.int32(ITER2), body, jnp.int32(0))
    plsc.subcore_barrier()

    for k in range(NCHUNK):
        pltpu.sync_copy(agg_sh.at[pl.ds(row0 + k * CHUNK, CHUNK)], bounce)
        pltpu.sync_copy(bounce, out_hbm.at[c, pl.ds(row0 + k * CHUNK, CHUNK)])


_sc_segment = pl.kernel(
    _sc_body,
    out_type=jax.ShapeDtypeStruct((NC, NPAD, 16), jnp.float32),
    mesh=plsc.VectorSubcoreMesh(core_axis_name="c", subcore_axis_name="s"),
    compiler_params=pltpu.CompilerParams(use_tc_tiling_on_sc=False,
                                         needs_layout_passes=False),
    scratch_types=[
        pltpu.VMEM_SHARED((NPAD, 16), jnp.float32),
        pltpu.VMEM((BLK,), jnp.int32),
        pltpu.VMEM((BLK,), jnp.int32),
        pltpu.VMEM((128,), jnp.int32),
        pltpu.VMEM((128,), jnp.int32),
        pltpu.VMEM((128,), jnp.int32),
        pltpu.VMEM((128,), jnp.int32),
        pltpu.VMEM((4 * BLK,), jnp.float32),
        pltpu.VMEM((4 * BLK,), jnp.float32),
        pltpu.VMEM((BLK, 16), jnp.float32),
        pltpu.VMEM((BLK, 16), jnp.float32),
        pltpu.VMEM((4, 16), jnp.float32),
        pltpu.VMEM((CHUNK, 16), jnp.float32),
        pltpu.SemaphoreType.DMA,
        pltpu.SemaphoreType.DMA,
        pltpu.SemaphoreType.DMA,
        pltpu.SemaphoreType.DMA,
    ],
)


def _upd_body(agg2_ref, x_ref, nz_ref, w2_ref, b2_ref, o_ref):
    agg = agg2_ref[0] + agg2_ref[1]
    h = jnp.tanh(jnp.dot(agg, w2_ref[...],
                         preferred_element_type=jnp.float32) + b2_ref[...])
    x = x_ref[...]
    col4 = x[:, 4]
    c_mask = col4 == 1.0
    moveable = jnp.logical_or(c_mask, col4 == 2.0)
    cm = c_mask.astype(jnp.float32)
    h = h * moveable.astype(jnp.float32)[:, None]
    acc = h[:, :2] * ACC_SCALE
    vel = jnp.clip(x[:, 2:4] + acc, jnp.float32(-MAX_VEL), jnp.float32(MAX_VEL))
    pos = x[:, :2] + vel
    pos = jnp.where(pos > 1.0, pos - 2.0, pos)
    pos = jnp.where(pos < -1.0, pos + 2.0, pos)
    new_v = jnp.where(moveable[:, None], vel, x[:, 2:4]) + nz_ref[...] * cm[:, None]
    new_p = jnp.where(moveable[:, None], pos, x[:, :2])
    energy = jnp.minimum(x[:, 5] - cm, 10.0)
    o_ref[...] = jnp.concatenate(
        [new_p, new_v, x[:, 4:5], energy[:, None], h[:, 2:]], axis=1)


def _update(agg2, x, nz, w2, b2):
    bn = 2000
    return pl.pallas_call(
        _upd_body,
        grid=(N // bn,),
        in_specs=[pl.BlockSpec((2, bn, 8), lambda i: (_z(), i, _z())),
                  pl.BlockSpec((bn, 11), lambda i: (i, _z())),
                  pl.BlockSpec((bn, 2), lambda i: (i, _z())),
                  pl.BlockSpec((8, 7), lambda i: (_z(), _z())),
                  pl.BlockSpec((1, 7), lambda i: (_z(), _z()))],
        out_specs=pl.BlockSpec((bn, 11), lambda i: (i, _z())),
        out_shape=jax.ShapeDtypeStruct((N, 11), jnp.float32),
    )(agg2, x, nz, w2, b2)


def kernel(x, edge_index, edge_attr, W1, b1, W2, b2):
    src = edge_index[0].astype(jnp.int32)
    dst = edge_index[1].astype(jnp.int32)
    x = x.astype(jnp.float32)
    edge_attr = edge_attr.astype(jnp.float32)
    W1 = W1.astype(jnp.float32)
    b1 = b1.astype(jnp.float32)
    W2 = W2.astype(jnp.float32)
    b2 = b2.astype(jnp.float32)
    pad8 = jnp.zeros((8,), jnp.float32)
    w1a = jnp.concatenate([W1[:8], jnp.zeros((8, 8), jnp.float32)], axis=1)
    w1b = jnp.concatenate([W1[8:], jnp.zeros((4, 8), jnp.float32)], axis=1)
    b1p = jnp.concatenate([b1, pad8])

    u = _compute_u(x, w1a, b1p[None, :])
    ea_flat = edge_attr.T.reshape(4 * E)
    zeros = jnp.zeros((CHUNK, 16), jnp.float32)

    agg2 = _sc_segment(u, ea_flat, src, dst, w1b, zeros)[:, :N, :8]

    nk = jax.random.key(42)
    xn = (jax.random.uniform(jax.random.fold_in(nk, 0), (N,), dtype=jnp.float32) * 2.0 - 1.0) * NOISE
    yn = (jax.random.uniform(jax.random.fold_in(nk, 1), (N,), dtype=jnp.float32) * 2.0 - 1.0) * NOISE
    upd = (jax.random.uniform(jax.random.fold_in(nk, 2), (N,), dtype=jnp.float32) > 0.5).astype(jnp.float32)
    nz = jnp.stack([xn * upd, yn * upd], axis=1)

    return _update(agg2, x, nz, W2, b2[None, :]).astype(jnp.float64)


# speedup vs baseline: 179.7423x; 2.1594x over previous
import jax
import jax.numpy as jnp
from jax import lax
from jax.experimental import pallas as pl
from jax.experimental.pallas import tpu as pltpu
from jax.experimental.pallas import tpu_sc as plsc


def _z():
    return jnp.int32(0)


N = 100000
E = 6400000
ACC_SCALE = 0.02
MAX_VEL = 0.02
NOISE = 0.002

NC = 2
NS = 16
BLK = 512
NW = NC * NS
NBLK = E // BLK
ITER2 = -(-NBLK // (2 * NW))
NPAD = 100096
ROWS_PER_TILE = NPAD // NS
CHUNK = 184
NCHUNK = ROWS_PER_TILE // CHUNK


def _u_body(x_ref, w_ref, b_ref, o_ref):
    o_ref[...] = jnp.dot(x_ref[:, :8], w_ref[...],
                         preferred_element_type=jnp.float32) + b_ref[...]


def _compute_u(x, w1a, b1):
    bn = 10000
    return pl.pallas_call(
        _u_body,
        grid=(N // bn,),
        in_specs=[pl.BlockSpec((bn, 11), lambda i: (i, _z())),
                  pl.BlockSpec((8, 16), lambda i: (_z(), _z())),
                  pl.BlockSpec((1, 16), lambda i: (_z(), _z()))],
        out_specs=pl.BlockSpec((bn, 16), lambda i: (i, _z())),
        out_shape=jax.ShapeDtypeStruct((N, 16), jnp.float32),
    )(x, w1a, b1)


def _sc_body(u_hbm, ea_hbm, src_hbm, dst_hbm, w1b_hbm, zeros_hbm, out_hbm,
             agg_sh,
             srcb0, srcb1, dst0, dst1, eab0, eab1,
             ug0, ug1, wv, bounce,
             sl0, sl1, sg0, sg1):
    c = lax.axis_index("c")
    s = lax.axis_index("s")
    w = s * NC + c

    row0 = s * ROWS_PER_TILE
    pltpu.sync_copy(zeros_hbm, bounce)
    for k in range(NCHUNK):
        pltpu.sync_copy(bounce, agg_sh.at[pl.ds(row0 + k * CHUNK, CHUNK)])
    pltpu.sync_copy(w1b_hbm, wv)
    plsc.subcore_barrier()

    w0 = wv[0, :]
    w1 = wv[1, :]
    w2 = wv[2, :]
    w3 = wv[3, :]

    bufs = ((srcb0, dst0, eab0, ug0, sl0, sg0),
            (srcb1, dst1, eab1, ug1, sl1, sg1))

    def issue_lin(g, b):
        srcb, dst, eab, ug, sl, sg = bufs[b]

        @pl.when(g < NBLK)
        def _():
            off = g * BLK
            pltpu.async_copy(src_hbm.at[pl.ds(off, BLK)], srcb, sl)
            for k in range(BLK // 128):
                pltpu.async_copy(dst_hbm.at[pl.ds(off + 128 * k, 128)],
                                 dst[k], sl)
            for f in range(4):
                pltpu.async_copy(ea_hbm.at[pl.ds(f * E + off, BLK)],
                                 eab.at[pl.ds(f * BLK, BLK)], sl)

    def wait_lin_issue_gather(g, b):
        srcb, dst, eab, ug, sl, sg = bufs[b]

        @pl.when(g < NBLK)
        def _():
            off = g * BLK
            pltpu.make_async_copy(src_hbm.at[pl.ds(off, BLK)], srcb, sl).wait()
            for k in range(BLK // 128):
                pltpu.make_async_copy(dst_hbm.at[pl.ds(off + 128 * k, 128)],
                                      dst[k], sl).wait()
            for f in range(4):
                pltpu.make_async_copy(ea_hbm.at[pl.ds(f * E + off, BLK)],
                                      eab.at[pl.ds(f * BLK, BLK)], sl).wait()
            for k in range(BLK // 128):
                pltpu.async_copy(u_hbm.at[srcb.at[pl.ds(128 * k, 128)]],
                                 ug.at[pl.ds(128 * k, 128)], sg)

    def compute_scatter(g, b):
        srcb, dst, eab, ug, sl, sg = bufs[b]

        @pl.when(g < NBLK)
        def _():
            for k in range(BLK // 128):
                pltpu.make_async_copy(u_hbm.at[srcb.at[pl.ds(128 * k, 128)]],
                                      ug.at[pl.ds(128 * k, 128)], sg).wait()
            @plsc.parallel_loop(jnp.int32(0), jnp.int32(BLK),
                                step=jnp.int32(16), unroll=2)
            def _grp(base):
                ea0 = eab[pl.ds(base, 16)]
                ea1 = eab[pl.ds(base + BLK, 16)]
                ea2 = eab[pl.ds(base + 2 * BLK, 16)]
                ea3 = eab[pl.ds(base + 3 * BLK, 16)]
                for jj in range(16):
                    idx = jnp.full((16,), jj, jnp.int32)
                    z = ug[base + jj, :]
                    z = z + ea0.at[idx].get(mode="promise_in_bounds") * w0
                    z = z + ea1.at[idx].get(mode="promise_in_bounds") * w1
                    z = z + ea2.at[idx].get(mode="promise_in_bounds") * w2
                    z = z + ea3.at[idx].get(mode="promise_in_bounds") * w3
                    e = jnp.exp(2.0 * z)
                    ug[base + jj, :] = 1.0 - 2.0 / (e + 1.0)
            for k in range(BLK // 128):
                pltpu.sync_copy(ug.at[pl.ds(128 * k, 128)],
                                agg_sh.at[dst[k]], add=True)

    def gblock(i2, b):
        return w + jnp.int32(2 * NW) * i2 + jnp.int32(b * NW)

    issue_lin(gblock(jnp.int32(0), 0), 0)

    def body(i2, carry):
        for b in (0, 1):
            g = gblock(i2, b)
            wait_lin_issue_gather(g, b)
            if b == 0:
                issue_lin(gblock(i2, 1), 1)
            else:
                issue_lin(gblock(i2 + 1, 0), 0)
            compute_scatter(g, b)
        return carry

    lax.fori_loop(jnp.int32(0), jnp.int32(ITER2), body, jnp.int32(0))
    plsc.subcore_barrier()

    for k in range(NCHUNK):
        pltpu.sync_copy(agg_sh.at[pl.ds(row0 + k * CHUNK, CHUNK)], bounce)
        pltpu.sync_copy(bounce, out_hbm.at[c, pl.ds(row0 + k * CHUNK, CHUNK)])


_sc_segment = pl.kernel(
    _sc_body,
    out_type=jax.ShapeDtypeStruct((NC, NPAD, 16), jnp.float32),
    mesh=plsc.VectorSubcoreMesh(core_axis_name="c", subcore_axis_name="s"),
    compiler_params=pltpu.CompilerParams(use_tc_tiling_on_sc=False,
                                         needs_layout_passes=False),
    scratch_types=[
        pltpu.VMEM_SHARED((NPAD, 16), jnp.float32),
        pltpu.VMEM((BLK,), jnp.int32),
        pltpu.VMEM((BLK,), jnp.int32),
        [pltpu.VMEM((128,), jnp.int32) for _ in range(BLK // 128)],
        [pltpu.VMEM((128,), jnp.int32) for _ in range(BLK // 128)],
        pltpu.VMEM((4 * BLK,), jnp.float32),
        pltpu.VMEM((4 * BLK,), jnp.float32),
        pltpu.VMEM((BLK, 16), jnp.float32),
        pltpu.VMEM((BLK, 16), jnp.float32),
        pltpu.VMEM((4, 16), jnp.float32),
        pltpu.VMEM((CHUNK, 16), jnp.float32),
        pltpu.SemaphoreType.DMA,
        pltpu.SemaphoreType.DMA,
        pltpu.SemaphoreType.DMA,
        pltpu.SemaphoreType.DMA,
    ],
)


def _upd_body(agg2_ref, x_ref, nz_ref, w2_ref, b2_ref, o_ref):
    agg = agg2_ref[0] + agg2_ref[1]
    h = jnp.tanh(jnp.dot(agg, w2_ref[...],
                         preferred_element_type=jnp.float32) + b2_ref[...])
    x = x_ref[...]
    col4 = x[:, 4]
    c_mask = col4 == 1.0
    moveable = jnp.logical_or(c_mask, col4 == 2.0)
    cm = c_mask.astype(jnp.float32)
    h = h * moveable.astype(jnp.float32)[:, None]
    acc = h[:, :2] * ACC_SCALE
    vel = jnp.clip(x[:, 2:4] + acc, jnp.float32(-MAX_VEL), jnp.float32(MAX_VEL))
    pos = x[:, :2] + vel
    pos = jnp.where(pos > 1.0, pos - 2.0, pos)
    pos = jnp.where(pos < -1.0, pos + 2.0, pos)
    new_v = jnp.where(moveable[:, None], vel, x[:, 2:4]) + nz_ref[...] * cm[:, None]
    new_p = jnp.where(moveable[:, None], pos, x[:, :2])
    energy = jnp.minimum(x[:, 5] - cm, 10.0)
    o_ref[...] = jnp.concatenate(
        [new_p, new_v, x[:, 4:5], energy[:, None], h[:, 2:]], axis=1)


def _update(agg2, x, nz, w2, b2):
    bn = 2000
    return pl.pallas_call(
        _upd_body,
        grid=(N // bn,),
        in_specs=[pl.BlockSpec((2, bn, 8), lambda i: (_z(), i, _z())),
                  pl.BlockSpec((bn, 11), lambda i: (i, _z())),
                  pl.BlockSpec((bn, 2), lambda i: (i, _z())),
                  pl.BlockSpec((8, 7), lambda i: (_z(), _z())),
                  pl.BlockSpec((1, 7), lambda i: (_z(), _z()))],
        out_specs=pl.BlockSpec((bn, 11), lambda i: (i, _z())),
        out_shape=jax.ShapeDtypeStruct((N, 11), jnp.float32),
    )(agg2, x, nz, w2, b2)


def kernel(x, edge_index, edge_attr, W1, b1, W2, b2):
    src = edge_index[0].astype(jnp.int32)
    dst = edge_index[1].astype(jnp.int32)
    x = x.astype(jnp.float32)
    edge_attr = edge_attr.astype(jnp.float32)
    W1 = W1.astype(jnp.float32)
    b1 = b1.astype(jnp.float32)
    W2 = W2.astype(jnp.float32)
    b2 = b2.astype(jnp.float32)
    pad8 = jnp.zeros((8,), jnp.float32)
    w1a = jnp.concatenate([W1[:8], jnp.zeros((8, 8), jnp.float32)], axis=1)
    w1b = jnp.concatenate([W1[8:], jnp.zeros((4, 8), jnp.float32)], axis=1)
    b1p = jnp.concatenate([b1, pad8])

    u = _compute_u(x, w1a, b1p[None, :])
    ea_flat = edge_attr.T.reshape(4 * E)
    zeros = jnp.zeros((CHUNK, 16), jnp.float32)

    agg2 = _sc_segment(u, ea_flat, src, dst, w1b, zeros)[:, :N, :8]

    nk = jax.random.key(42)
    xn = (jax.random.uniform(jax.random.fold_in(nk, 0), (N,), dtype=jnp.float32) * 2.0 - 1.0) * NOISE
    yn = (jax.random.uniform(jax.random.fold_in(nk, 1), (N,), dtype=jnp.float32) * 2.0 - 1.0) * NOISE
    upd = (jax.random.uniform(jax.random.fold_in(nk, 2), (N,), dtype=jnp.float32) > 0.5).astype(jnp.float32)
    nz = jnp.stack([xn * upd, yn * upd], axis=1)

    return _update(agg2, x, nz, W2, b2[None, :]).astype(jnp.float64)
